# baseline (speedup 1.0000x reference)
import functools

import jax
import jax.numpy as jnp
from jax import lax
from jax.experimental import pallas as pl
from jax.experimental.pallas import tpu as pltpu
from jax.experimental.pallas import tpu_sc as plsc

D = 1024
FF = 4096
E = 8
K = 2

BLK = 512
FFT = 512
J = FF // FFT
T = 4096
A = T * K
NB = A // BLK + E
P = NB * BLK

NC, NS, L = 2, 16, 16
NW = NC * NS

TB = 1024


def _router_body(x_ref, rw_ref, rb_ref, e_ref, w_ref):
    logits = jnp.dot(x_ref[...], rw_ref[...], preferred_element_type=jnp.float32)
    logits = logits + rb_ref[...]
    idx8 = lax.broadcasted_iota(jnp.int32, (TB, E), 1)
    m0 = jnp.max(logits, axis=-1, keepdims=True)
    e0 = jnp.min(jnp.where(logits == m0, idx8, E), axis=-1, keepdims=True)
    masked = jnp.where(idx8 == e0, -jnp.inf, logits)
    m1 = jnp.max(masked, axis=-1, keepdims=True)
    e1 = jnp.min(jnp.where(masked == m1, idx8, E), axis=-1, keepdims=True)
    w0 = jax.nn.sigmoid(m0 - m1)
    e_ref[...] = jnp.concatenate([e0, e1], axis=1)
    w_ref[...] = jnp.concatenate([w0, 1.0 - w0], axis=1)


def _router(x_flat, rw, rb2):
    return pl.pallas_call(
        _router_body,
        grid=(T // TB,),
        in_specs=[
            pl.BlockSpec((TB, D), lambda i: (i, 0)),
            pl.BlockSpec((D, E), lambda i: (0, 0)),
            pl.BlockSpec((1, E), lambda i: (0, 0)),
        ],
        out_specs=[
            pl.BlockSpec((TB, K), lambda i: (i, 0)),
            pl.BlockSpec((TB, K), lambda i: (i, 0)),
        ],
        out_shape=[
            jax.ShapeDtypeStruct((T, K), jnp.int32),
            jax.ShapeDtypeStruct((T, K), jnp.float32),
        ],
    )(x_flat, rw, rb2)


_R_CHK = A // NW
_R_NCH = _R_CHK // L
_R_SCAN = A // L


def _route_body(e_hbm, w_hbm, pos_out, src_out, gw_out, cnt_out,
                e_all, w_buf, pos_buf, tok_buf, cnt_v, sem0, sem1):
    wid = lax.axis_index("s") * NC + lax.axis_index("c")
    pltpu.sync_copy(e_hbm, e_all)
    pltpu.sync_copy(w_hbm.at[wid], w_buf)
    lane = lax.iota(jnp.int32, 16)
    zero16 = jnp.zeros((16,), jnp.int32)
    my_first = wid * _R_NCH

    def scan_body(i, carry):
        cnt, pref = carry
        snap = (i == my_first).astype(jnp.int32)
        pref = pref + snap * (cnt - pref)
        v = e_all[pl.ds(i * L, L)]
        for e in range(E):
            c = plsc.all_reduce_population_count(v == e)
            cnt = cnt + jnp.where(lane == e, c, zero16)
        return (cnt, pref)

    totals, pref = lax.fori_loop(0, _R_SCAN, scan_body, (zero16, zero16))

    nbb = ((totals + (BLK - 1)) // BLK) * BLK
    pad_off = plsc.cumsum(nbb) - nbb
    base_vec = pad_off + pref

    @pl.when(wid == 0)
    def _():
        cnt_v[...] = totals
        pltpu.sync_copy(cnt_v, cnt_out)

    base = [jnp.sum(jnp.where(lane == e, base_vec, zero16)) for e in range(E)]
    for ch in range(_R_NCH):
        v = e_all[pl.ds((my_first + ch) * L, L)]
        pos = zero16
        for e in range(E):
            m = v == e
            incl = plsc.cumsum(m.astype(jnp.int32))
            pos = jnp.where(m, base[e] + (incl - 1), pos)
            base[e] = base[e] + jnp.max(incl)
        r, c0 = ch // 8, (ch % 8) * L
        pos_buf[r, pl.ds(c0, L)] = pos
        tok_buf[r, pl.ds(c0, L)] = (wid * _R_CHK + ch * L + lane) // K
    pltpu.sync_copy(pos_buf, pos_out.at[wid])
    for r in range(2):
        pltpu.async_copy(tok_buf.at[r], src_out.at[pos_buf.at[r]], sem0).wait()
        pltpu.async_copy(w_buf.at[r], gw_out.at[pos_buf.at[r]], sem1).wait()


def _route(e_flat, w3):
    mesh = plsc.VectorSubcoreMesh(core_axis_name="c", subcore_axis_name="s")
    return pl.kernel(
        _route_body,
        out_type=[
            jax.ShapeDtypeStruct((NW, 2, 128), jnp.int32),
            jax.ShapeDtypeStruct((P,), jnp.int32),
            jax.ShapeDtypeStruct((P,), jnp.float32),
            jax.ShapeDtypeStruct((L,), jnp.int32),
        ],
        mesh=mesh,
        scratch_types=[
            pltpu.VMEM((A,), jnp.int32),
            pltpu.VMEM((2, 128), jnp.float32),
            pltpu.VMEM((2, 128), jnp.int32),
            pltpu.VMEM((2, 128), jnp.int32),
            pltpu.VMEM((L,), jnp.int32),
            pltpu.SemaphoreType.DMA,
            pltpu.SemaphoreType.DMA,
        ],
        compiler_params=pltpu.CompilerParams(needs_layout_passes=False),
    )(e_flat, w3)


_G_SLOTS = P // NW
_G_NBUF = 4
_G_CH = 48
_G_N = _G_SLOTS // _G_CH
_G_W = D // 2


def _dispatch_body(src_hbm, x_hbm, xg_hbm, idx_v, *bufs_sems):
    bufs = bufs_sems[:_G_NBUF]
    gsems = bufs_sems[_G_NBUF:2 * _G_NBUF]
    ssems = bufs_sems[2 * _G_NBUF:]
    wid = lax.axis_index("s") * NC + lax.axis_index("c")
    base = wid * _G_SLOTS
    pltpu.sync_copy(src_hbm.at[pl.ds(base, _G_SLOTS)], idx_v)
    for q in range(_G_SLOTS // L):
        sl = pl.ds(q * L, L)
        idx_v[sl] = jnp.minimum(jnp.maximum(idx_v[sl], 0), T - 1)

    def gather(k):
        return pltpu.async_copy(
            x_hbm.at[idx_v.at[pl.ds(k * _G_CH, _G_CH)]],
            bufs[k % _G_NBUF], gsems[k % _G_NBUF])

    g = [None] * _G_N
    s = [None] * _G_N
    for k in range(_G_NBUF - 1):
        g[k] = gather(k)
    for k in range(_G_N):
        kn = k + _G_NBUF - 1
        if kn < _G_N:
            if k >= 1:
                s[k - 1].wait()
            g[kn] = gather(kn)
        g[k].wait()
        s[k] = pltpu.async_copy(
            bufs[k % _G_NBUF],
            xg_hbm.at[pl.ds(base + k * _G_CH, _G_CH)],
            ssems[k % _G_NBUF])
    for k in range(_G_N - _G_NBUF, _G_N):
        s[k].wait()


def _dispatch(src_token, x_flat):
    mesh = plsc.VectorSubcoreMesh(core_axis_name="c", subcore_axis_name="s")
    return pl.kernel(
        _dispatch_body,
        out_type=jax.ShapeDtypeStruct((P, _G_W), jnp.int32),
        mesh=mesh,
        scratch_types=[
            pltpu.VMEM((_G_SLOTS,), jnp.int32),
            *[pltpu.VMEM((_G_CH, _G_W), jnp.int32) for _ in range(_G_NBUF)],
            *[pltpu.SemaphoreType.DMA for _ in range(2 * _G_NBUF)],
        ],
    )(src_token, x_flat)


def _ffn_body(be, bv, xg_ref, w1_ref, b1_ref, w2_ref, b2_ref, gw_ref,
              y_ref, acc_ref):
    b = pl.program_id(0)
    j = pl.program_id(1)
    valid = bv[b] == 1

    @pl.when(valid)
    def _():
        xb = xg_ref[...].astype(jnp.float32)
        h = jnp.dot(xb, w1_ref[0], preferred_element_type=jnp.float32)
        h = h + b1_ref[0]
        h = 0.5 * h * (1.0 + lax.erf(h * (2.0 ** -0.5)))
        part = jnp.dot(h, w2_ref[0], preferred_element_type=jnp.float32)

        @pl.when(j == 0)
        def _():
            acc_ref[...] = part

        @pl.when(j > 0)
        def _():
            acc_ref[...] += part

    @pl.when(valid & (j == J - 1))
    def _():
        y_ref[...] = (acc_ref[...] + b2_ref[0]) * gw_ref[...]


def _ffn(block_e, block_valid, xg, w1, b1, w2, b2, gw2):
    grid_spec = pltpu.PrefetchScalarGridSpec(
        num_scalar_prefetch=2,
        grid=(NB, J),
        in_specs=[
            pl.BlockSpec((BLK, D), lambda b, j, be, bv: (jnp.where(bv[b] == 1, b, 0), 0)),
            pl.BlockSpec((1, D, FFT), lambda b, j, be, bv: (be[b], 0, jnp.where(bv[b] == 1, j, 0))),
            pl.BlockSpec((1, 1, FFT), lambda b, j, be, bv: (be[b], 0, jnp.where(bv[b] == 1, j, 0))),
            pl.BlockSpec((1, FFT, D), lambda b, j, be, bv: (be[b], jnp.where(bv[b] == 1, j, 0), 0)),
            pl.BlockSpec((1, 1, D), lambda b, j, be, bv: (be[b], 0, 0)),
            pl.BlockSpec((BLK, 1), lambda b, j, be, bv: (jnp.where(bv[b] == 1, b, 0), 0)),
        ],
        out_specs=pl.BlockSpec((BLK, D), lambda b, j, be, bv: (b, 0)),
        scratch_shapes=[pltpu.VMEM((BLK, D), jnp.float32)],
    )
    return pl.pallas_call(
        _ffn_body,
        grid_spec=grid_spec,
        out_shape=jax.ShapeDtypeStruct((P, D), jnp.float32),
        compiler_params=pltpu.CompilerParams(
            dimension_semantics=("arbitrary", "arbitrary")),
    )(block_e, block_valid, xg, w1, b1.reshape(E, 1, FF), w2,
      b2.reshape(E, 1, D), gw2)


_C_TOK = T // NW
_C_CH = 16


_C_N = _C_TOK // _C_CH


def _combine_body(y_hbm, pos_hbm, out_hbm, pos_v, rows0, rows1, out0, out1,
                  gs0, gs1, ss0, ss1):
    wid = lax.axis_index("s") * NC + lax.axis_index("c")
    pltpu.sync_copy(pos_hbm.at[pl.ds(wid * K * _C_TOK, K * _C_TOK)], pos_v)
    rbufs, obufs = (rows0, rows1), (out0, out1)
    gsems, ssems = (gs0, gs1), (ss0, ss1)
    g = [None] * _C_N
    s = [None] * _C_N
    g[0] = pltpu.async_copy(
        y_hbm.at[pos_v.at[pl.ds(0, K * _C_CH)]], rbufs[0], gsems[0])
    for k in range(_C_N):
        cur = k & 1
        if k + 1 < _C_N:
            g[k + 1] = pltpu.async_copy(
                y_hbm.at[pos_v.at[pl.ds((k + 1) * K * _C_CH, K * _C_CH)]],
                rbufs[(k + 1) & 1], gsems[(k + 1) & 1])
        g[k].wait()
        if k >= 2:
            s[k - 2].wait()
        rows_v, out_v = rbufs[cur], obufs[cur]

        def body(i, _):
            for dd in range(D // L):
                sl = pl.ds(dd * L, L)
                out_v[i, sl] = rows_v[2 * i, sl] + rows_v[2 * i + 1, sl]
            return 0

        lax.fori_loop(0, _C_CH, body, 0)
        s[k] = pltpu.async_copy(
            out_v, out_hbm.at[pl.ds(wid * _C_TOK + k * _C_CH, _C_CH)],
            ssems[cur])
    s[_C_N - 2].wait()
    s[_C_N - 1].wait()


def _combine(y, pos_flat):
    mesh = plsc.VectorSubcoreMesh(core_axis_name="c", subcore_axis_name="s")
    return pl.kernel(
        _combine_body,
        out_type=jax.ShapeDtypeStruct((T, D), jnp.float32),
        mesh=mesh,
        scratch_types=[
            pltpu.VMEM((K * _C_TOK,), jnp.int32),
            pltpu.VMEM((K * _C_CH, D), jnp.float32),
            pltpu.VMEM((K * _C_CH, D), jnp.float32),
            pltpu.VMEM((_C_CH, D), jnp.float32),
            pltpu.VMEM((_C_CH, D), jnp.float32),
            pltpu.SemaphoreType.DMA,
            pltpu.SemaphoreType.DMA,
            pltpu.SemaphoreType.DMA,
            pltpu.SemaphoreType.DMA,
        ],
    )(y, pos_flat)


def kernel(x, router_w, router_b, w1, b1, w2, b2):
    B, S, _ = x.shape
    x_flat = x.reshape(T, D)

    e01, w01 = _router(x_flat, router_w, router_b.reshape(1, E))

    pos3, src_token, gw, counts16 = _route(
        e01.reshape(A), w01.reshape(NW, 2, 128))
    counts = counts16[:E]
    nb = (counts + BLK - 1) // BLK
    cum_nb = jnp.cumsum(nb)
    bidx = jnp.arange(NB, dtype=jnp.int32)
    block_e = jnp.minimum(
        jnp.searchsorted(cum_nb, bidx, side="right"), E - 1).astype(jnp.int32)
    block_valid = (bidx < cum_nb[-1]).astype(jnp.int32)

    x32 = lax.bitcast_convert_type(
        x_flat.astype(jnp.bfloat16).reshape(T, _G_W, 2), jnp.int32)
    xg = lax.bitcast_convert_type(
        _dispatch(src_token, x32), jnp.bfloat16).reshape(P, D)
    y = _ffn(block_e, block_valid, xg, w1, b1, w2, b2, gw.reshape(P, 1))
    out_flat = _combine(y, pos3.reshape(A))
    return out_flat.reshape(B, S, D)


# speedup vs baseline: 1.4317x; 1.4317x over previous
import functools

import jax
import jax.numpy as jnp
from jax import lax
from jax.experimental import pallas as pl
from jax.experimental.pallas import tpu as pltpu
from jax.experimental.pallas import tpu_sc as plsc

D = 1024
FF = 4096
E = 8
K = 2

BLK = 512
FFT = 512
J = FF // FFT
T = 4096
A = T * K
NB = A // BLK + E
P = NB * BLK

NC, NS, L = 2, 16, 16
NW = NC * NS

TB = 1024


def _router_body(x_ref, rw_ref, rb_ref, e_ref, w_ref):
    logits = jnp.dot(x_ref[...], rw_ref[...], preferred_element_type=jnp.float32)
    logits = logits + rb_ref[...]
    idx8 = lax.broadcasted_iota(jnp.int32, (TB, E), 1)
    m0 = jnp.max(logits, axis=-1, keepdims=True)
    e0 = jnp.min(jnp.where(logits == m0, idx8, E), axis=-1, keepdims=True)
    masked = jnp.where(idx8 == e0, -jnp.inf, logits)
    m1 = jnp.max(masked, axis=-1, keepdims=True)
    e1 = jnp.min(jnp.where(masked == m1, idx8, E), axis=-1, keepdims=True)
    w0 = jax.nn.sigmoid(m0 - m1)
    e_ref[...] = jnp.concatenate([e0, e1], axis=1)
    w_ref[...] = jnp.concatenate([w0, 1.0 - w0], axis=1)


def _router(x_flat, rw, rb2):
    return pl.pallas_call(
        _router_body,
        grid=(T // TB,),
        in_specs=[
            pl.BlockSpec((TB, D), lambda i: (i, 0)),
            pl.BlockSpec((D, E), lambda i: (0, 0)),
            pl.BlockSpec((1, E), lambda i: (0, 0)),
        ],
        out_specs=[
            pl.BlockSpec((TB, K), lambda i: (i, 0)),
            pl.BlockSpec((TB, K), lambda i: (i, 0)),
        ],
        out_shape=[
            jax.ShapeDtypeStruct((T, K), jnp.int32),
            jax.ShapeDtypeStruct((T, K), jnp.float32),
        ],
    )(x_flat, rw, rb2)


_R_CHK = A // NW
_R_NCH = _R_CHK // L
_R_SCAN = A // L


def _route_body(e_hbm, w_hbm, pos_out, src_out, gw_out, cnt_out,
                e_all, w_buf, pos_buf, tok_buf, cnt_v, sem0, sem1):
    wid = lax.axis_index("s") * NC + lax.axis_index("c")
    pltpu.sync_copy(e_hbm, e_all)
    pltpu.sync_copy(w_hbm.at[wid], w_buf)
    lane = lax.iota(jnp.int32, 16)
    zero16 = jnp.zeros((16,), jnp.int32)
    my_first = wid * _R_NCH

    def scan_body(i, carry):
        cnt, pref = carry
        snap = (i == my_first).astype(jnp.int32)
        pref = pref + snap * (cnt - pref)
        v = e_all[pl.ds(i * L, L)]
        for e in range(E):
            c = plsc.all_reduce_population_count(v == e)
            cnt = cnt + jnp.where(lane == e, c, zero16)
        return (cnt, pref)

    totals, pref = lax.fori_loop(0, _R_SCAN, scan_body, (zero16, zero16))

    nbb = ((totals + (BLK - 1)) // BLK) * BLK
    pad_off = plsc.cumsum(nbb) - nbb
    base_vec = pad_off + pref

    @pl.when(wid == 0)
    def _():
        cnt_v[...] = totals
        pltpu.sync_copy(cnt_v, cnt_out)

    base = [jnp.sum(jnp.where(lane == e, base_vec, zero16)) for e in range(E)]
    for ch in range(_R_NCH):
        v = e_all[pl.ds((my_first + ch) * L, L)]
        pos = zero16
        for e in range(E):
            m = v == e
            incl = plsc.cumsum(m.astype(jnp.int32))
            pos = jnp.where(m, base[e] + (incl - 1), pos)
            base[e] = base[e] + jnp.max(incl)
        r, c0 = ch // 8, (ch % 8) * L
        pos_buf[r, pl.ds(c0, L)] = pos
        tok_buf[r, pl.ds(c0, L)] = (wid * _R_CHK + ch * L + lane) // K
    pltpu.sync_copy(pos_buf, pos_out.at[wid])
    for r in range(2):
        pltpu.async_copy(tok_buf.at[r], src_out.at[pos_buf.at[r]], sem0).wait()
        pltpu.async_copy(w_buf.at[r], gw_out.at[pos_buf.at[r]], sem1).wait()


def _route(e_flat, w3):
    mesh = plsc.VectorSubcoreMesh(core_axis_name="c", subcore_axis_name="s")
    return pl.kernel(
        _route_body,
        out_type=[
            jax.ShapeDtypeStruct((NW, 2, 128), jnp.int32),
            jax.ShapeDtypeStruct((P,), jnp.int32),
            jax.ShapeDtypeStruct((P,), jnp.float32),
            jax.ShapeDtypeStruct((L,), jnp.int32),
        ],
        mesh=mesh,
        scratch_types=[
            pltpu.VMEM((A,), jnp.int32),
            pltpu.VMEM((2, 128), jnp.float32),
            pltpu.VMEM((2, 128), jnp.int32),
            pltpu.VMEM((2, 128), jnp.int32),
            pltpu.VMEM((L,), jnp.int32),
            pltpu.SemaphoreType.DMA,
            pltpu.SemaphoreType.DMA,
        ],
        compiler_params=pltpu.CompilerParams(needs_layout_passes=False),
    )(e_flat, w3)


_G_SLOTS = P // NW
_G_NBUF = 4
_G_CH = 24
_G_N = _G_SLOTS // _G_CH


def _dispatch_body(src_hbm, x_hbm, xg_hbm, idx_v, *bufs_sems):
    bufs = bufs_sems[:_G_NBUF]
    gsems = bufs_sems[_G_NBUF:2 * _G_NBUF]
    ssems = bufs_sems[2 * _G_NBUF:]
    wid = lax.axis_index("s") * NC + lax.axis_index("c")
    base = wid * _G_SLOTS
    pltpu.sync_copy(src_hbm.at[pl.ds(base, _G_SLOTS)], idx_v)
    for q in range(_G_SLOTS // L):
        sl = pl.ds(q * L, L)
        idx_v[sl] = jnp.minimum(jnp.maximum(idx_v[sl], 0), T - 1)

    def gather(k):
        return pltpu.async_copy(
            x_hbm.at[idx_v.at[pl.ds(k * _G_CH, _G_CH)]],
            bufs[k % _G_NBUF], gsems[k % _G_NBUF])

    g = [None] * _G_N
    s = [None] * _G_N
    for k in range(_G_NBUF - 1):
        g[k] = gather(k)
    for k in range(_G_N):
        kn = k + _G_NBUF - 1
        if kn < _G_N:
            if k >= 1:
                s[k - 1].wait()
            g[kn] = gather(kn)
        g[k].wait()
        s[k] = pltpu.async_copy(
            bufs[k % _G_NBUF],
            xg_hbm.at[pl.ds(base + k * _G_CH, _G_CH)],
            ssems[k % _G_NBUF])
    for k in range(_G_N - _G_NBUF, _G_N):
        s[k].wait()


def _dispatch(src_token, x_flat):
    mesh = plsc.VectorSubcoreMesh(core_axis_name="c", subcore_axis_name="s")
    return pl.kernel(
        _dispatch_body,
        out_type=jax.ShapeDtypeStruct((P, D), jnp.float32),
        mesh=mesh,
        scratch_types=[
            pltpu.VMEM((_G_SLOTS,), jnp.int32),
            *[pltpu.VMEM((_G_CH, D), jnp.float32) for _ in range(_G_NBUF)],
            *[pltpu.SemaphoreType.DMA for _ in range(2 * _G_NBUF)],
        ],
    )(src_token, x_flat)


def _ffn_body(be, bv, xg_ref, w1_ref, b1_ref, w2_ref, b2_ref, gw_ref,
              y_ref, acc_ref):
    b = pl.program_id(0)
    j = pl.program_id(1)
    valid = bv[b] == 1

    @pl.when(valid)
    def _():
        h = jnp.dot(xg_ref[...], w1_ref[0], preferred_element_type=jnp.float32)
        h = h + b1_ref[0]
        h = 0.5 * h * (1.0 + lax.erf(h * (2.0 ** -0.5)))
        part = jnp.dot(h, w2_ref[0], preferred_element_type=jnp.float32)

        @pl.when(j == 0)
        def _():
            acc_ref[...] = part

        @pl.when(j > 0)
        def _():
            acc_ref[...] += part

    @pl.when(valid & (j == J - 1))
    def _():
        y_ref[...] = (acc_ref[...] + b2_ref[0]) * gw_ref[...]


def _ffn(block_e, block_valid, xg, w1, b1, w2, b2, gw2):
    grid_spec = pltpu.PrefetchScalarGridSpec(
        num_scalar_prefetch=2,
        grid=(NB, J),
        in_specs=[
            pl.BlockSpec((BLK, D), lambda b, j, be, bv: (jnp.where(bv[b] == 1, b, 0), 0)),
            pl.BlockSpec((1, D, FFT), lambda b, j, be, bv: (be[b], 0, jnp.where(bv[b] == 1, j, 0))),
            pl.BlockSpec((1, 1, FFT), lambda b, j, be, bv: (be[b], 0, jnp.where(bv[b] == 1, j, 0))),
            pl.BlockSpec((1, FFT, D), lambda b, j, be, bv: (be[b], jnp.where(bv[b] == 1, j, 0), 0)),
            pl.BlockSpec((1, 1, D), lambda b, j, be, bv: (be[b], 0, 0)),
            pl.BlockSpec((BLK, 1), lambda b, j, be, bv: (jnp.where(bv[b] == 1, b, 0), 0)),
        ],
        out_specs=pl.BlockSpec((BLK, D), lambda b, j, be, bv: (b, 0)),
        scratch_shapes=[pltpu.VMEM((BLK, D), jnp.float32)],
    )
    return pl.pallas_call(
        _ffn_body,
        grid_spec=grid_spec,
        out_shape=jax.ShapeDtypeStruct((P, D), jnp.float32),
        compiler_params=pltpu.CompilerParams(
            dimension_semantics=("arbitrary", "arbitrary")),
    )(block_e, block_valid, xg, w1, b1.reshape(E, 1, FF), w2,
      b2.reshape(E, 1, D), gw2)


_C_TOK = T // NW
_C_CH = 16


_C_N = _C_TOK // _C_CH


def _combine_body(y_hbm, pos_hbm, out_hbm, pos_v, rows0, rows1, out0, out1,
                  gs0, gs1, ss0, ss1):
    wid = lax.axis_index("s") * NC + lax.axis_index("c")
    pltpu.sync_copy(pos_hbm.at[pl.ds(wid * K * _C_TOK, K * _C_TOK)], pos_v)
    rbufs, obufs = (rows0, rows1), (out0, out1)
    gsems, ssems = (gs0, gs1), (ss0, ss1)
    g = [None] * _C_N
    s = [None] * _C_N
    g[0] = pltpu.async_copy(
        y_hbm.at[pos_v.at[pl.ds(0, K * _C_CH)]], rbufs[0], gsems[0])
    for k in range(_C_N):
        cur = k & 1
        if k + 1 < _C_N:
            g[k + 1] = pltpu.async_copy(
                y_hbm.at[pos_v.at[pl.ds((k + 1) * K * _C_CH, K * _C_CH)]],
                rbufs[(k + 1) & 1], gsems[(k + 1) & 1])
        g[k].wait()
        if k >= 2:
            s[k - 2].wait()
        rows_v, out_v = rbufs[cur], obufs[cur]

        def body(i, _):
            for dd in range(D // L):
                sl = pl.ds(dd * L, L)
                out_v[i, sl] = rows_v[2 * i, sl] + rows_v[2 * i + 1, sl]
            return 0

        lax.fori_loop(0, _C_CH, body, 0)
        s[k] = pltpu.async_copy(
            out_v, out_hbm.at[pl.ds(wid * _C_TOK + k * _C_CH, _C_CH)],
            ssems[cur])
    s[_C_N - 2].wait()
    s[_C_N - 1].wait()


def _combine(y, pos_flat):
    mesh = plsc.VectorSubcoreMesh(core_axis_name="c", subcore_axis_name="s")
    return pl.kernel(
        _combine_body,
        out_type=jax.ShapeDtypeStruct((T, D), jnp.float32),
        mesh=mesh,
        scratch_types=[
            pltpu.VMEM((K * _C_TOK,), jnp.int32),
            pltpu.VMEM((K * _C_CH, D), jnp.float32),
            pltpu.VMEM((K * _C_CH, D), jnp.float32),
            pltpu.VMEM((_C_CH, D), jnp.float32),
            pltpu.VMEM((_C_CH, D), jnp.float32),
            pltpu.SemaphoreType.DMA,
            pltpu.SemaphoreType.DMA,
            pltpu.SemaphoreType.DMA,
            pltpu.SemaphoreType.DMA,
        ],
    )(y, pos_flat)


def kernel(x, router_w, router_b, w1, b1, w2, b2):
    B, S, _ = x.shape
    x_flat = x.reshape(T, D)

    e01, w01 = _router(x_flat, router_w, router_b.reshape(1, E))

    pos3, src_token, gw, counts16 = _route(
        e01.reshape(A), w01.reshape(NW, 2, 128))
    counts = counts16[:E]
    nb = (counts + BLK - 1) // BLK
    cum_nb = jnp.cumsum(nb)
    bidx = jnp.arange(NB, dtype=jnp.int32)
    block_e = jnp.minimum(
        jnp.searchsorted(cum_nb, bidx, side="right"), E - 1).astype(jnp.int32)
    block_valid = (bidx < cum_nb[-1]).astype(jnp.int32)

    xg = _dispatch(src_token, x_flat)
    y = _ffn(block_e, block_valid, xg, w1, b1, w2, b2, gw.reshape(P, 1))
    out_flat = _combine(y, pos3.reshape(A))
    return out_flat.reshape(B, S, D)
